# baseline (speedup 1.0000x reference)
import functools
import math

import jax
import jax.numpy as jnp
from jax import lax
from jax.experimental import pallas as pl
from jax.experimental.pallas import tpu as pltpu
from jax.experimental.pallas import tpu_sc as plsc

_D = 128
_S = 200
_B = 1024
_SCALE = math.sqrt(_D)
_LANES = 16

_info = plsc.get_sparse_core_info()
_NC = _info.num_cores
_NS = _info.num_subcores
_NW = _NC * _NS
_ROWS = _B * _S
_RPW = _ROWS // _NW

_CHUNK = 40
_NBUF = 12
_NCH = _RPW // _CHUNK
_POSF = _S // _CHUNK

_mesh = plsc.VectorSubcoreMesh(core_axis_name="c", subcore_axis_name="s")


@functools.partial(
    pl.kernel,
    mesh=_mesh,
    out_type=jax.ShapeDtypeStruct((_ROWS, _D), jnp.float32),
    scratch_types=[
        pltpu.VMEM((_RPW,), jnp.int32),
        pltpu.VMEM((_S, _D), jnp.float32),
        pltpu.VMEM((_NBUF * _CHUNK, _D), jnp.float32),
        pltpu.SemaphoreType.DMA((_NBUF,)),
        pltpu.SemaphoreType.DMA((_NBUF,)),
    ],
)
def _emb(x_hbm, tok_hbm, pos_hbm, out_hbm, idx_v, pos_v, bufs, gsem, osem):
    wid = lax.axis_index("s") * _NC + lax.axis_index("c")
    wbase = wid * _RPW

    pltpu.sync_copy(x_hbm.at[pl.ds(wbase, _RPW)], idx_v)
    pltpu.sync_copy(pos_hbm.at[pl.ds(0, _S)], pos_v)

    def gather_start(c, b):
        pltpu.async_copy(
            tok_hbm.at[idx_v.at[pl.ds(c * _CHUNK, _CHUNK)]],
            bufs.at[pl.ds(b * _CHUNK, _CHUNK)],
            gsem.at[b],
        )

    def gather_wait(b):
        pltpu.make_async_copy(
            tok_hbm.at[pl.ds(0, _CHUNK)], bufs.at[pl.ds(0, _CHUNK)], gsem.at[b]
        ).wait()

    def out_start(c, b):
        pltpu.async_copy(
            bufs.at[pl.ds(b * _CHUNK, _CHUNK)],
            out_hbm.at[pl.ds(wbase + c * _CHUNK, _CHUNK)],
            osem.at[b],
        )

    def out_wait(b):
        pltpu.make_async_copy(
            tok_hbm.at[pl.ds(0, _CHUNK)], bufs.at[pl.ds(0, _CHUNK)], osem.at[b]
        ).wait()

    for i in range(_NBUF - 1):
        gather_start(i, i)

    def chunk_body(c, carry):
        b = lax.rem(c, _NBUF)
        gather_wait(b)
        pbase = lax.rem(c, _POSF) * _CHUNK

        @plsc.parallel_loop(0, _CHUNK, unroll=4)
        def row_body(r):
            rr = b * _CHUNK + r
            for j in range(_D // _LANES):
                sl = pl.ds(j * _LANES, _LANES)
                bufs[rr, sl] = bufs[rr, sl] * _SCALE + pos_v[pbase + r, sl]

        out_start(c, b)

        @pl.when(c + _NBUF - 1 < _NCH)
        def _prefetch():
            cp = c + _NBUF - 1
            b2 = lax.rem(cp, _NBUF)

            @pl.when(c >= 1)
            def _drain():
                out_wait(b2)

            gather_start(cp, b2)

        return carry

    lax.fori_loop(0, _NCH, chunk_body, 0)
    for b in range(_NBUF):
        out_wait(b)


def kernel(x, token_table, pos_table):
    idx = x.reshape(-1).astype(jnp.int32)
    out = _emb(idx, token_table, pos_table)
    return out.reshape(x.shape[0], x.shape[1], _D)


# speedup vs baseline: 1.0164x; 1.0164x over previous
import functools
import math

import jax
import jax.numpy as jnp
from jax import lax
from jax.experimental import pallas as pl
from jax.experimental.pallas import tpu as pltpu
from jax.experimental.pallas import tpu_sc as plsc

_D = 128
_S = 200
_B = 1024
_SCALE = math.sqrt(_D)
_LANES = 16

_info = plsc.get_sparse_core_info()
_NC = _info.num_cores
_NS = _info.num_subcores
_NW = _NC * _NS
_ROWS = _B * _S
_RPW = _ROWS // _NW

_CHUNK = 40
_NBUF = 8
_NCH = _RPW // _CHUNK
_POSF = _S // _CHUNK

_mesh = plsc.VectorSubcoreMesh(core_axis_name="c", subcore_axis_name="s")


@functools.partial(
    pl.kernel,
    mesh=_mesh,
    out_type=jax.ShapeDtypeStruct((_ROWS, _D), jnp.float32),
    scratch_types=[
        pltpu.VMEM((_RPW,), jnp.int32),
        pltpu.VMEM((_S, _D), jnp.float32),
        pltpu.VMEM((_NBUF * _CHUNK, _D), jnp.float32),
        pltpu.SemaphoreType.DMA((_NBUF,)),
        pltpu.SemaphoreType.DMA((_NBUF,)),
        pltpu.SemaphoreType.DMA,
    ],
)
def _emb(x_hbm, tok_hbm, pos_hbm, out_hbm, idx_v, pos_v, bufs, gsem, osem,
         psem):
    wid = lax.axis_index("s") * _NC + lax.axis_index("c")
    wbase = wid * _RPW

    pos_cp = pltpu.make_async_copy(pos_hbm.at[pl.ds(0, _S)], pos_v, psem)
    pos_cp.start()
    pltpu.sync_copy(x_hbm.at[pl.ds(wbase, _RPW)], idx_v)

    def gather_start(c, b):
        pltpu.async_copy(
            tok_hbm.at[idx_v.at[pl.ds(c * _CHUNK, _CHUNK)]],
            bufs.at[pl.ds(b * _CHUNK, _CHUNK)],
            gsem.at[b],
        )

    def gather_wait(b):
        pltpu.make_async_copy(
            tok_hbm.at[pl.ds(0, _CHUNK)], bufs.at[pl.ds(0, _CHUNK)], gsem.at[b]
        ).wait()

    def out_start(c, b):
        pltpu.async_copy(
            bufs.at[pl.ds(b * _CHUNK, _CHUNK)],
            out_hbm.at[pl.ds(wbase + c * _CHUNK, _CHUNK)],
            osem.at[b],
        )

    def out_wait(b):
        pltpu.make_async_copy(
            tok_hbm.at[pl.ds(0, _CHUNK)], bufs.at[pl.ds(0, _CHUNK)], osem.at[b]
        ).wait()

    for i in range(_NBUF - 1):
        gather_start(i, i)
    pos_cp.wait()

    def chunk_body(c, carry):
        b = lax.rem(c, _NBUF)
        gather_wait(b)
        pbase = lax.rem(c, _POSF) * _CHUNK

        @plsc.parallel_loop(0, _CHUNK, unroll=4)
        def row_body(r):
            rr = b * _CHUNK + r
            for j in range(_D // _LANES):
                sl = pl.ds(j * _LANES, _LANES)
                bufs[rr, sl] = bufs[rr, sl] * _SCALE + pos_v[pbase + r, sl]

        out_start(c, b)

        @pl.when(c + _NBUF - 1 < _NCH)
        def _prefetch():
            cp = c + _NBUF - 1
            b2 = lax.rem(cp, _NBUF)

            @pl.when(c >= 1)
            def _drain():
                out_wait(b2)

            gather_start(cp, b2)

        return carry

    lax.fori_loop(0, _NCH, chunk_body, 0)
    for b in range(_NBUF):
        out_wait(b)


def kernel(x, token_table, pos_table):
    idx = x.reshape(-1).astype(jnp.int32)
    out = _emb(idx, token_table, pos_table)
    return out.reshape(x.shape[0], x.shape[1], _D)
